# Initial kernel scaffold; baseline (speedup 1.0000x reference)
import jax
import jax.numpy as jnp
from jax.experimental import pallas as pl


def kernel(x, ei, W1, b1, W2, b2):
    raise NotImplementedError("write your pallas kernel here")


# speedup vs baseline: 6.7974x; 6.7974x over previous
import functools

import jax
import jax.numpy as jnp
from jax import lax
from jax.experimental import pallas as pl
from jax.experimental.pallas import tpu as pltpu
from jax.experimental.pallas import tpu_sc as plsc

N = 10000
FEAT = 128
HID = 32
EMB = 16

NC = 2
NS = 16
NW = NC * NS
W = 128
CHUNK = 128
CPW = 80
E_PAD = NW * CPW * CHUNK
EPW = CPW * CHUNK
N_PAD = 10112
ROWS_PER_SUB = N_PAD // NS

_mesh = plsc.VectorSubcoreMesh(
    core_axis_name="c", subcore_axis_name="s", num_cores=NC, num_subcores=NS)


def _sc_degree_body(dst_hbm, zeros_hbm, ones_hbm, out_hbm, dstrow, ones_v, deg_sp):
    c = lax.axis_index("c")
    s = lax.axis_index("s")
    wid = s * NC + c
    rows = pl.ds(s * ROWS_PER_SUB, ROWS_PER_SUB)
    pltpu.sync_copy(zeros_hbm.at[rows], deg_sp.at[rows])
    pltpu.sync_copy(ones_hbm, ones_v)
    plsc.subcore_barrier()

    def body(j, carry):
        pltpu.sync_copy(dst_hbm.at[pl.ds(wid * EPW + j * CHUNK, CHUNK)], dstrow)
        pltpu.sync_copy(ones_v, deg_sp.at[dstrow], add=True)
        return carry

    lax.fori_loop(0, CPW, body, 0)
    plsc.subcore_barrier()
    pltpu.sync_copy(deg_sp.at[rows], out_hbm.at[c, rows])


_sc_degree = pl.kernel(
    _sc_degree_body,
    out_type=jax.ShapeDtypeStruct((NC, N_PAD, W), jnp.float32),
    mesh=_mesh,
    scratch_types=[
        pltpu.VMEM((CHUNK,), jnp.int32),
        pltpu.VMEM((CHUNK, W), jnp.float32),
        pltpu.VMEM_SHARED((N_PAD, W), jnp.float32),
    ],
)


def _sc_agg_body(hs_hbm, src_hbm, dst_hbm, zeros_hbm, out_hbm,
                 srcrow, dstrow, rows_v, agg_sp, sem):
    c = lax.axis_index("c")
    s = lax.axis_index("s")
    wid = s * NC + c
    rows = pl.ds(s * ROWS_PER_SUB, ROWS_PER_SUB)
    pltpu.sync_copy(zeros_hbm.at[rows], agg_sp.at[rows])
    plsc.subcore_barrier()

    def body(j, carry):
        base = wid * EPW + j * CHUNK
        pltpu.sync_copy(src_hbm.at[pl.ds(base, CHUNK)], srcrow)
        pltpu.sync_copy(dst_hbm.at[pl.ds(base, CHUNK)], dstrow)
        pltpu.async_copy(hs_hbm.at[srcrow], rows_v, sem).wait()
        pltpu.sync_copy(rows_v, agg_sp.at[dstrow], add=True)
        return carry

    lax.fori_loop(0, CPW, body, 0)
    plsc.subcore_barrier()
    pltpu.sync_copy(agg_sp.at[rows], out_hbm.at[c, rows])


_sc_agg = pl.kernel(
    _sc_agg_body,
    out_type=jax.ShapeDtypeStruct((NC, N_PAD, W), jnp.float32),
    mesh=_mesh,
    scratch_types=[
        pltpu.VMEM((CHUNK,), jnp.int32),
        pltpu.VMEM((CHUNK,), jnp.int32),
        pltpu.VMEM((CHUNK, W), jnp.float32),
        pltpu.VMEM_SHARED((N_PAD, W), jnp.float32),
        pltpu.SemaphoreType.DMA,
    ],
)


def _dinv_from(degp_ref):
    deg = degp_ref[0, :, 0:1] + degp_ref[1, :, 0:1] + 1.0
    return lax.rsqrt(deg)


def _tc_a_body(x_ref, w1_ref, degp_ref, hs_ref):
    dinv = _dinv_from(degp_ref)
    h = jnp.dot(x_ref[...], w1_ref[...], preferred_element_type=jnp.float32)
    hs_ref[...] = jnp.concatenate(
        [h * dinv, jnp.zeros((h.shape[0], W - HID), jnp.float32)], axis=1)


def _tc_b_body(aggp_ref, hs1_ref, degp_ref, b1_ref, w2_ref, hs2_ref):
    dinv = _dinv_from(degp_ref)
    agg = (aggp_ref[0] + aggp_ref[1] + hs1_ref[...])[:, :HID]
    out1 = jax.nn.relu(agg * dinv + b1_ref[...])
    h2 = jnp.dot(out1, w2_ref[...], preferred_element_type=jnp.float32)
    hs2_ref[...] = jnp.concatenate(
        [h2 * dinv, jnp.zeros((h2.shape[0], W - EMB), jnp.float32)], axis=1)


def _tc_c_body(aggp_ref, hs2_ref, degp_ref, b2_ref, z_ref):
    dinv = _dinv_from(degp_ref)
    agg = (aggp_ref[0] + aggp_ref[1] + hs2_ref[...])[:, :EMB]
    z_ref[...] = agg * dinv + b2_ref[...]


def kernel(x, ei, W1, b1, W2, b2):
    x_pad = jnp.pad(x, ((0, N_PAD - N), (0, 0)))
    ei_pad = jnp.pad(ei, ((0, 0), (0, E_PAD - ei.shape[1])), constant_values=N)
    src_flat = ei_pad[0]
    dst_flat = ei_pad[1]
    zeros_w = jnp.zeros((N_PAD, W), jnp.float32)
    ones_w = jnp.ones((CHUNK, W), jnp.float32)

    degp = _sc_degree(dst_flat, zeros_w, ones_w)

    hs1 = pl.pallas_call(
        _tc_a_body,
        out_shape=jax.ShapeDtypeStruct((N_PAD, W), jnp.float32),
    )(x_pad, W1, degp)

    aggp1 = _sc_agg(hs1, src_flat, dst_flat, zeros_w)

    hs2 = pl.pallas_call(
        _tc_b_body,
        out_shape=jax.ShapeDtypeStruct((N_PAD, W), jnp.float32),
    )(aggp1, hs1, degp, b1, W2)

    aggp2 = _sc_agg(hs2, src_flat, dst_flat, zeros_w)

    z_pad = pl.pallas_call(
        _tc_c_body,
        out_shape=jax.ShapeDtypeStruct((N_PAD, EMB), jnp.float32),
    )(aggp2, hs2, degp, b2)

    return z_pad[:N]


# speedup vs baseline: 6.9237x; 1.0186x over previous
import functools

import jax
import jax.numpy as jnp
from jax import lax
from jax.experimental import pallas as pl
from jax.experimental.pallas import tpu as pltpu
from jax.experimental.pallas import tpu_sc as plsc

N = 10000
FEAT = 128
HID = 32
EMB = 16

NC = 2
NS = 16
NW = NC * NS
W = 128
CHUNK = 128
CPW = 80
E_PAD = NW * CPW * CHUNK
EPW = CPW * CHUNK
N_PAD = 10112
ROWS_PER_SUB = N_PAD // NS

_mesh = plsc.VectorSubcoreMesh(
    core_axis_name="c", subcore_axis_name="s", num_cores=NC, num_subcores=NS)


def _sc_degree_body(dst_hbm, zeros_hbm, ones_hbm, out_hbm, dstrow, ones_v, deg_sp):
    c = lax.axis_index("c")
    s = lax.axis_index("s")
    wid = s * NC + c
    rows = pl.ds(s * ROWS_PER_SUB, ROWS_PER_SUB)
    pltpu.sync_copy(zeros_hbm.at[rows], deg_sp.at[rows])
    pltpu.sync_copy(ones_hbm, ones_v)
    plsc.subcore_barrier()

    def body(j, carry):
        pltpu.sync_copy(dst_hbm.at[pl.ds(wid * EPW + j * CHUNK, CHUNK)], dstrow)
        pltpu.sync_copy(ones_v, deg_sp.at[dstrow], add=True)
        return carry

    lax.fori_loop(0, CPW, body, 0)
    plsc.subcore_barrier()
    pltpu.sync_copy(deg_sp.at[rows], out_hbm.at[c, rows])


_sc_degree = pl.kernel(
    _sc_degree_body,
    out_type=jax.ShapeDtypeStruct((NC, N_PAD, W), jnp.float32),
    mesh=_mesh,
    scratch_types=[
        pltpu.VMEM((CHUNK,), jnp.int32),
        pltpu.VMEM((CHUNK, W), jnp.float32),
        pltpu.VMEM_SHARED((N_PAD, W), jnp.float32),
    ],
)


def _sc_agg_body(hs_hbm, src_hbm, dst_hbm, zeros_hbm, out_hbm,
                 srcA, srcB, dstA, dstB, rowsA, rowsB,
                 semGA, semGB, agg_sp):
    c = lax.axis_index("c")
    s = lax.axis_index("s")
    wid = s * NC + c
    rows = pl.ds(s * ROWS_PER_SUB, ROWS_PER_SUB)
    pltpu.sync_copy(zeros_hbm.at[rows], agg_sp.at[rows])
    plsc.subcore_barrier()

    def body(g, carry):
        base = wid * EPW + g * (2 * CHUNK)
        pltpu.sync_copy(src_hbm.at[pl.ds(base, CHUNK)], srcA)
        pltpu.sync_copy(src_hbm.at[pl.ds(base + CHUNK, CHUNK)], srcB)
        pltpu.sync_copy(dst_hbm.at[pl.ds(base, CHUNK)], dstA)
        pltpu.sync_copy(dst_hbm.at[pl.ds(base + CHUNK, CHUNK)], dstB)
        ga = pltpu.async_copy(hs_hbm.at[srcA], rowsA, semGA)
        gb = pltpu.async_copy(hs_hbm.at[srcB], rowsB, semGB)
        ga.wait()
        pltpu.sync_copy(rowsA, agg_sp.at[dstA], add=True)
        gb.wait()
        pltpu.sync_copy(rowsB, agg_sp.at[dstB], add=True)
        return carry

    lax.fori_loop(0, CPW // 2, body, 0)
    plsc.subcore_barrier()
    pltpu.sync_copy(agg_sp.at[rows], out_hbm.at[c, rows])


_sc_agg = pl.kernel(
    _sc_agg_body,
    out_type=jax.ShapeDtypeStruct((NC, N_PAD, W), jnp.float32),
    mesh=_mesh,
    scratch_types=[
        pltpu.VMEM((CHUNK,), jnp.int32),
        pltpu.VMEM((CHUNK,), jnp.int32),
        pltpu.VMEM((CHUNK,), jnp.int32),
        pltpu.VMEM((CHUNK,), jnp.int32),
        pltpu.VMEM((CHUNK, W), jnp.float32),
        pltpu.VMEM((CHUNK, W), jnp.float32),
        pltpu.SemaphoreType.DMA,
        pltpu.SemaphoreType.DMA,
        pltpu.VMEM_SHARED((N_PAD, W), jnp.float32),
    ],
)


def _dinv_from(degp_ref):
    deg = degp_ref[0, :, 0:1] + degp_ref[1, :, 0:1] + 1.0
    return lax.rsqrt(deg)


def _tc_a_body(x_ref, w1_ref, degp_ref, hs_ref):
    dinv = _dinv_from(degp_ref)
    h = jnp.dot(x_ref[...], w1_ref[...], preferred_element_type=jnp.float32)
    hs_ref[...] = jnp.concatenate(
        [h * dinv, jnp.zeros((h.shape[0], W - HID), jnp.float32)], axis=1)


def _tc_b_body(aggp_ref, hs1_ref, degp_ref, b1_ref, w2_ref, hs2_ref):
    dinv = _dinv_from(degp_ref)
    agg = (aggp_ref[0] + aggp_ref[1] + hs1_ref[...])[:, :HID]
    out1 = jax.nn.relu(agg * dinv + b1_ref[...])
    h2 = jnp.dot(out1, w2_ref[...], preferred_element_type=jnp.float32)
    hs2_ref[...] = jnp.concatenate(
        [h2 * dinv, jnp.zeros((h2.shape[0], W - EMB), jnp.float32)], axis=1)


def _tc_c_body(aggp_ref, hs2_ref, degp_ref, b2_ref, z_ref):
    dinv = _dinv_from(degp_ref)
    agg = (aggp_ref[0] + aggp_ref[1] + hs2_ref[...])[:, :EMB]
    z_ref[...] = agg * dinv + b2_ref[...]


def kernel(x, ei, W1, b1, W2, b2):
    x_pad = jnp.pad(x, ((0, N_PAD - N), (0, 0)))
    ei_pad = jnp.pad(ei, ((0, 0), (0, E_PAD - ei.shape[1])), constant_values=N)
    src_flat = ei_pad[0]
    dst_flat = ei_pad[1]
    zeros_w = jnp.zeros((N_PAD, W), jnp.float32)
    ones_w = jnp.ones((CHUNK, W), jnp.float32)

    degp = _sc_degree(dst_flat, zeros_w, ones_w)

    hs1 = pl.pallas_call(
        _tc_a_body,
        out_shape=jax.ShapeDtypeStruct((N_PAD, W), jnp.float32),
    )(x_pad, W1, degp)

    aggp1 = _sc_agg(hs1, src_flat, dst_flat, zeros_w)

    hs2 = pl.pallas_call(
        _tc_b_body,
        out_shape=jax.ShapeDtypeStruct((N_PAD, W), jnp.float32),
    )(aggp1, hs1, degp, b1, W2)

    aggp2 = _sc_agg(hs2, src_flat, dst_flat, zeros_w)

    z_pad = pl.pallas_call(
        _tc_c_body,
        out_shape=jax.ShapeDtypeStruct((N_PAD, EMB), jnp.float32),
    )(aggp2, hs2, degp, b2)

    return z_pad[:N]
